# baseline (speedup 1.0000x reference)
import jax
import jax.numpy as jnp
from jax import lax
from jax.experimental import pallas as pl
from jax.experimental.pallas import tpu as pltpu
from jax.experimental.pallas import tpu_sc as plsc

NUM_LABELS = 1000000
EMBED = 32
SLOTS = 5
BATCH = 16384
OUT_D = SLOTS * EMBED

_info = plsc.get_sparse_core_info()
NC, NS = _info.num_cores, _info.num_subcores
NW = NC * NS
B_PER_W = BATCH // NW
BCHUNK = 8
NCH = B_PER_W // BCHUNK
LOOKUPS = BCHUNK * SLOTS


def _body(x_hbm, table_hbm, wfull_hbm, out_hbm, blocks, rows2, w_v, x_v, sem):
    wid = lax.axis_index("s") * NC + lax.axis_index("c")
    base = wid * B_PER_W

    pltpu.sync_copy(wfull_hbm, w_v)
    wvec = [w_v[pl.ds(16 * k, 16)] for k in range(2 * SLOTS)]

    @pl.loop(0, NCH)
    def _(c):
        pltpu.sync_copy(x_hbm.at[pl.ds(base + c * BCHUNK, BCHUNK)], x_v)

        @pl.loop(0, BCHUNK)
        def _(g):
            row = x_v[g, pl.ds(0, 16)]
            for j in range(SLOTS):
                idx = row[j]
                j8 = pl.multiple_of((idx >> 3) << 3, 8)
                pltpu.async_copy(
                    table_hbm.at[pl.ds(j8, 8)], blocks.at[g * SLOTS + j], sem
                )

        for k in range(LOOKUPS):
            pltpu.make_async_copy(
                table_hbm.at[pl.ds(0, 8)], blocks.at[k], sem
            ).wait()

        @pl.loop(0, BCHUNK)
        def _(g):
            row = x_v[g, pl.ds(0, 16)]
            for j in range(SLOTS):
                jm = row[j] & 7
                for h in range(2):
                    rows2[g, pl.ds(j * EMBED + 16 * h, 16)] = (
                        blocks[g * SLOTS + j, jm, pl.ds(16 * h, 16)]
                        * wvec[2 * j + h]
                    )

        pltpu.sync_copy(rows2, out_hbm.at[pl.ds(base + c * BCHUNK, BCHUNK)])


@jax.jit
def _gather_scale(x, table, wfull):
    mesh = plsc.VectorSubcoreMesh(core_axis_name="c", subcore_axis_name="s")
    return pl.kernel(
        _body,
        out_type=jax.ShapeDtypeStruct((BATCH, OUT_D), jnp.float32),
        mesh=mesh,
        scratch_types=[
            pltpu.VMEM((LOOKUPS, 8, EMBED), jnp.float32),
            pltpu.VMEM((BCHUNK, OUT_D), jnp.float32),
            pltpu.VMEM((2 * SLOTS * 16,), jnp.float32),
            pltpu.VMEM((BCHUNK, SLOTS), jnp.int32),
            pltpu.SemaphoreType.DMA,
        ],
        compiler_params=pltpu.CompilerParams(use_tc_tiling_on_sc=True),
    )(x, table, wfull)


def kernel(x, table, weight):
    wfull = jnp.repeat(weight.astype(jnp.float32), EMBED)
    return _gather_scale(x.astype(jnp.int32), table, wfull)
